# baseline (speedup 1.0000x reference)
import jax
import jax.numpy as jnp
from jax.experimental import pallas as pl
from jax.experimental.pallas import tpu as pltpu
from functools import partial


def _body(np_ref, feat_ref, pfT_ref, w1aT_ref, wtailT_ref, w2T_ref,
          b2_ref, out_ref, ph_ref, *, n_patches):
    ph_ref[:, :n_patches] = jnp.dot(
        w1aT_ref[...], pfT_ref[...],
        preferred_element_type=jnp.float32).astype(jnp.bfloat16)
    ph_ref[:, n_patches:] = wtailT_ref[...]

    idx = np_ref[0, 0, :]
    blk = idx.shape[0]
    onehotT = (idx[None, :] == jax.lax.broadcasted_iota(
        jnp.int32, (n_patches, blk), 0)).astype(jnp.bfloat16)
    featT = jnp.transpose(feat_ref[0])
    augT = jnp.concatenate(
        [onehotT, featT,
         jnp.ones((1, blk), jnp.bfloat16),
         jnp.zeros((3, blk), jnp.bfloat16)], axis=0)
    hT = jnp.maximum(jnp.dot(ph_ref[...], augT,
                             preferred_element_type=jnp.float32), 0.0)
    out_ref[0] = jnp.dot(w2T_ref[...], hT.astype(jnp.bfloat16),
                         preferred_element_type=jnp.float32) + b2_ref[...]


def kernel(mixer_x, features, node_patch, W1, b1, W2, b2):
    b, t, p, f = mixer_x.shape
    n = features.shape[0]
    tf = t * f
    in_dim = W1.shape[0]
    horizon = W2.shape[1]

    pfT = mixer_x.transpose(0, 1, 3, 2).reshape(tf, p)
    w1aT = W1[:tf].T
    wtailT = jnp.concatenate(
        [W1[tf:], b1.reshape(1, in_dim), jnp.zeros((3, in_dim), jnp.float32)],
        axis=0).T.astype(jnp.bfloat16)
    w2T = W2.T.astype(jnp.bfloat16)
    b2c = b2.reshape(horizon, 1)

    blk = 20000
    grid = n // blk
    np3 = node_patch.astype(jnp.int32).reshape(grid, 1, blk)
    featn = features.astype(jnp.bfloat16).reshape(grid, blk, t)

    out3 = pl.pallas_call(
        partial(_body, n_patches=p),
        grid=(grid,),
        in_specs=[
            pl.BlockSpec((1, 1, blk), lambda i: (i, 0, 0)),
            pl.BlockSpec((1, blk, t), lambda i: (i, 0, 0)),
            pl.BlockSpec((tf, p), lambda i: (0, 0)),
            pl.BlockSpec((in_dim, tf), lambda i: (0, 0)),
            pl.BlockSpec((in_dim, 16), lambda i: (0, 0)),
            pl.BlockSpec((horizon, in_dim), lambda i: (0, 0)),
            pl.BlockSpec((horizon, 1), lambda i: (0, 0)),
        ],
        out_specs=pl.BlockSpec((1, horizon, blk), lambda i: (i, 0, 0)),
        out_shape=jax.ShapeDtypeStruct((grid, horizon, blk), jnp.float32),
        scratch_shapes=[pltpu.VMEM((in_dim, p + 16), jnp.bfloat16)],
    )(np3, featn, pfT, w1aT, wtailT, w2T, b2c)
    return out3.transpose(0, 2, 1).reshape(1, n, horizon)


# speedup vs baseline: 1.2356x; 1.2356x over previous
import jax
import jax.numpy as jnp
from jax.experimental import pallas as pl
from jax.experimental.pallas import tpu as pltpu
from functools import partial


def _body(np_ref, featT_ref, pfT_ref, w1aT_ref, wtailT_ref, w2T_ref,
          b2_ref, out_ref, ph_ref, *, n_patches):
    ph_ref[:, :n_patches] = jnp.dot(
        w1aT_ref[...], pfT_ref[...],
        preferred_element_type=jnp.float32).astype(jnp.bfloat16)
    ph_ref[:, n_patches:] = wtailT_ref[...]

    idx = np_ref[0, 0, :]
    blk = idx.shape[0]
    onehotT = (idx[None, :] == jax.lax.broadcasted_iota(
        jnp.int32, (n_patches, blk), 0)).astype(jnp.bfloat16)
    augT = jnp.concatenate(
        [onehotT, featT_ref[0],
         jnp.ones((1, blk), jnp.bfloat16),
         jnp.zeros((3, blk), jnp.bfloat16)], axis=0)
    hT = jnp.maximum(jnp.dot(ph_ref[...], augT,
                             preferred_element_type=jnp.float32), 0.0)
    out_ref[0] = jnp.dot(w2T_ref[...], hT.astype(jnp.bfloat16),
                         preferred_element_type=jnp.float32) + b2_ref[...]


def kernel(mixer_x, features, node_patch, W1, b1, W2, b2):
    b, t, p, f = mixer_x.shape
    n = features.shape[0]
    tf = t * f
    in_dim = W1.shape[0]
    horizon = W2.shape[1]

    pfT = mixer_x.transpose(0, 1, 3, 2).reshape(tf, p)
    w1aT = W1[:tf].T
    wtailT = jnp.concatenate(
        [W1[tf:], b1.reshape(1, in_dim), jnp.zeros((3, in_dim), jnp.float32)],
        axis=0).T.astype(jnp.bfloat16)
    w2T = W2.T.astype(jnp.bfloat16)
    b2c = b2.reshape(horizon, 1)

    blk = 25000
    grid = n // blk
    np3 = node_patch.astype(jnp.int32).reshape(grid, 1, blk)
    featT = features.astype(jnp.bfloat16).T.reshape(t, grid, blk)
    featT = featT.transpose(1, 0, 2)

    out3 = pl.pallas_call(
        partial(_body, n_patches=p),
        grid=(grid,),
        in_specs=[
            pl.BlockSpec((1, 1, blk), lambda i: (i, 0, 0)),
            pl.BlockSpec((1, t, blk), lambda i: (i, 0, 0)),
            pl.BlockSpec((tf, p), lambda i: (0, 0)),
            pl.BlockSpec((in_dim, tf), lambda i: (0, 0)),
            pl.BlockSpec((in_dim, 16), lambda i: (0, 0)),
            pl.BlockSpec((horizon, in_dim), lambda i: (0, 0)),
            pl.BlockSpec((horizon, 1), lambda i: (0, 0)),
        ],
        out_specs=pl.BlockSpec((1, horizon, blk), lambda i: (i, 0, 0)),
        out_shape=jax.ShapeDtypeStruct((grid, horizon, blk), jnp.float32),
        scratch_shapes=[pltpu.VMEM((in_dim, p + 16), jnp.bfloat16)],
    )(np3, featT, pfT, w1aT, wtailT, w2T, b2c)
    return out3.transpose(0, 2, 1).reshape(1, n, horizon)


# speedup vs baseline: 1.2402x; 1.0037x over previous
import jax
import jax.numpy as jnp
from jax.experimental import pallas as pl
from jax.experimental.pallas import tpu as pltpu
from functools import partial


def _body(np_ref, featT_ref, pfT_ref, w1aT_ref, wtailT_ref, w2T_ref,
          b2_ref, out_ref, ph_ref, *, n_patches):
    ph_ref[:, :n_patches] = jnp.dot(
        w1aT_ref[...], pfT_ref[...],
        preferred_element_type=jnp.float32).astype(jnp.bfloat16)
    ph_ref[:, n_patches:] = wtailT_ref[...]

    idx = np_ref[0, 0, :]
    blk = idx.shape[0]
    onehotT = (idx[None, :] == jax.lax.broadcasted_iota(
        jnp.int32, (n_patches, blk), 0)).astype(jnp.bfloat16)
    augT = jnp.concatenate(
        [onehotT, featT_ref[0],
         jnp.ones((1, blk), jnp.bfloat16),
         jnp.zeros((3, blk), jnp.bfloat16)], axis=0)
    hT = jnp.maximum(jnp.dot(ph_ref[...], augT,
                             preferred_element_type=jnp.float32), 0.0)
    out_ref[0] = jnp.dot(w2T_ref[...], hT.astype(jnp.bfloat16),
                         preferred_element_type=jnp.float32) + b2_ref[...]


def kernel(mixer_x, features, node_patch, W1, b1, W2, b2):
    b, t, p, f = mixer_x.shape
    n = features.shape[0]
    tf = t * f
    in_dim = W1.shape[0]
    horizon = W2.shape[1]

    pfT = mixer_x.transpose(0, 1, 3, 2).reshape(tf, p)
    w1aT = W1[:tf].T
    wtailT = jnp.concatenate(
        [W1[tf:], b1.reshape(1, in_dim), jnp.zeros((3, in_dim), jnp.float32)],
        axis=0).T.astype(jnp.bfloat16)
    w2T = W2.T.astype(jnp.bfloat16)
    b2c = b2.reshape(horizon, 1)

    blk = 50000
    grid = n // blk
    np3 = node_patch.astype(jnp.int32).reshape(grid, 1, blk)
    featT = features.astype(jnp.bfloat16).T.reshape(t, grid, blk)
    featT = featT.transpose(1, 0, 2)

    out3 = pl.pallas_call(
        partial(_body, n_patches=p),
        grid=(grid,),
        in_specs=[
            pl.BlockSpec((1, 1, blk), lambda i: (i, 0, 0)),
            pl.BlockSpec((1, t, blk), lambda i: (i, 0, 0)),
            pl.BlockSpec((tf, p), lambda i: (0, 0)),
            pl.BlockSpec((in_dim, tf), lambda i: (0, 0)),
            pl.BlockSpec((in_dim, 16), lambda i: (0, 0)),
            pl.BlockSpec((horizon, in_dim), lambda i: (0, 0)),
            pl.BlockSpec((horizon, 1), lambda i: (0, 0)),
        ],
        out_specs=pl.BlockSpec((1, horizon, blk), lambda i: (i, 0, 0)),
        out_shape=jax.ShapeDtypeStruct((grid, horizon, blk), jnp.float32),
        scratch_shapes=[pltpu.VMEM((in_dim, p + 16), jnp.bfloat16)],
    )(np3, featT, pfT, w1aT, wtailT, w2T, b2c)
    return out3.transpose(0, 2, 1).reshape(1, n, horizon)


# speedup vs baseline: 1.2524x; 1.0099x over previous
import jax
import jax.numpy as jnp
from jax.experimental import pallas as pl
from jax.experimental.pallas import tpu as pltpu
from functools import partial


def _body(np_ref, featT_ref, pfT_ref, w1aT_ref, wtailT_ref, w2T_ref,
          b2_ref, out_ref, ph_ref, *, n_patches):
    ph_ref[:, :n_patches] = jnp.dot(
        w1aT_ref[...], pfT_ref[...],
        preferred_element_type=jnp.float32).astype(jnp.bfloat16)
    ph_ref[:, n_patches:] = wtailT_ref[...]

    idx = np_ref[0, 0, :]
    blk = idx.shape[0]
    onehotT = (idx[None, :] == jax.lax.broadcasted_iota(
        jnp.int32, (n_patches, blk), 0)).astype(jnp.bfloat16)
    augT = jnp.concatenate(
        [onehotT, featT_ref[0],
         jnp.ones((1, blk), jnp.bfloat16),
         jnp.zeros((3, blk), jnp.bfloat16)], axis=0)
    hT = jnp.maximum(jnp.dot(ph_ref[...], augT,
                             preferred_element_type=jnp.float32), 0.0)
    out_ref[0] = (jnp.dot(w2T_ref[...], hT.astype(jnp.bfloat16),
                          preferred_element_type=jnp.float32)
                  + b2_ref[...]).astype(jnp.bfloat16)


def kernel(mixer_x, features, node_patch, W1, b1, W2, b2):
    b, t, p, f = mixer_x.shape
    n = features.shape[0]
    tf = t * f
    in_dim = W1.shape[0]
    horizon = W2.shape[1]

    pfT = mixer_x.transpose(0, 1, 3, 2).reshape(tf, p)
    w1aT = W1[:tf].T
    wtailT = jnp.concatenate(
        [W1[tf:], b1.reshape(1, in_dim), jnp.zeros((3, in_dim), jnp.float32)],
        axis=0).T.astype(jnp.bfloat16)
    w2T = W2.T.astype(jnp.bfloat16)
    b2c = b2.reshape(horizon, 1)

    blk = 25000
    grid = n // blk
    np3 = node_patch.astype(jnp.int32).reshape(grid, 1, blk)
    featT = features.astype(jnp.bfloat16).T.reshape(t, grid, blk)
    featT = featT.transpose(1, 0, 2)

    out3 = pl.pallas_call(
        partial(_body, n_patches=p),
        grid=(grid,),
        in_specs=[
            pl.BlockSpec((1, 1, blk), lambda i: (i, 0, 0)),
            pl.BlockSpec((1, t, blk), lambda i: (i, 0, 0)),
            pl.BlockSpec((tf, p), lambda i: (0, 0)),
            pl.BlockSpec((in_dim, tf), lambda i: (0, 0)),
            pl.BlockSpec((in_dim, 16), lambda i: (0, 0)),
            pl.BlockSpec((horizon, in_dim), lambda i: (0, 0)),
            pl.BlockSpec((horizon, 1), lambda i: (0, 0)),
        ],
        out_specs=pl.BlockSpec((1, horizon, blk), lambda i: (i, 0, 0)),
        out_shape=jax.ShapeDtypeStruct((grid, horizon, blk), jnp.bfloat16),
        scratch_shapes=[pltpu.VMEM((in_dim, p + 16), jnp.bfloat16)],
    )(np3, featT, pfT, w1aT, wtailT, w2T, b2c)
    return out3.transpose(0, 2, 1).astype(jnp.float32).reshape(1, n, horizon)
